# Initial kernel scaffold; baseline (speedup 1.0000x reference)
import jax
import jax.numpy as jnp
from jax.experimental import pallas as pl


def kernel(node_state, edge_weight, W, edge_source, edge_target):
    raise NotImplementedError("write your pallas kernel here")


# speedup vs baseline: 118.3264x; 118.3264x over previous
import functools

import jax
import jax.numpy as jnp
from jax import lax
from jax.experimental import pallas as pl
from jax.experimental.pallas import tpu as pltpu
from jax.experimental.pallas import tpu_sc as plsc

N_NODES = 100000
N_PAD = 100352
N_EDGES = 6400000
NC, NS = 2, 16
NW = NC * NS
EPW = N_EDGES // NW
C = 8000
NCHUNK = EPW // C
RSTG = N_PAD // NS
NPW = N_PAD // NW

_mesh = plsc.VectorSubcoreMesh(
    core_axis_name="c", subcore_axis_name="s", num_cores=NC, num_subcores=NS
)


def _iota16():
    return lax.broadcasted_iota(jnp.int32, (16,), 0)


@functools.partial(
    pl.kernel,
    out_type=(
        jax.ShapeDtypeStruct((NC * N_PAD,), jnp.float32),
        jax.ShapeDtypeStruct((NC * N_PAD,), jnp.float32),
    ),
    mesh=_mesh,
    scratch_types=[
        pltpu.VMEM_SHARED((N_PAD,), jnp.float32),
        pltpu.VMEM_SHARED((N_PAD,), jnp.float32),
        pltpu.VMEM_SHARED((N_PAD,), jnp.float32),
        pltpu.VMEM_SHARED((N_PAD,), jnp.float32),
        pltpu.VMEM((C,), jnp.int32),
        pltpu.VMEM((C,), jnp.int32),
        pltpu.VMEM((C,), jnp.float32),
        pltpu.VMEM((C,), jnp.float32),
        pltpu.VMEM((C,), jnp.float32),
        pltpu.VMEM((C,), jnp.float32),
        pltpu.VMEM((C,), jnp.float32),
        pltpu.VMEM((64,), jnp.float32),
        pltpu.SemaphoreType.DMA,
        pltpu.SemaphoreType.DMA,
    ],
)
def _scatter_kernel(n0_hbm, n1_hbm, wb_hbm, src_hbm, tgt_hbm, wgt_hbm,
                    out0_hbm, out1_hbm,
                    n0_sh, n1_sh, a0_sh, a1_sh,
                    src_v, tgt_v, w_v, s0_v, s1_v, m0_v, m1_v, wb_v,
                    sem0, sem1):
    cid = lax.axis_index("c")
    sid = lax.axis_index("s")
    wid = cid * NS + sid
    fz = jnp.zeros((16,), jnp.float32)

    ro = sid * RSTG
    pltpu.sync_copy(n0_hbm.at[pl.ds(ro, RSTG)], s0_v.at[pl.ds(0, RSTG)])
    pltpu.sync_copy(s0_v.at[pl.ds(0, RSTG)], n0_sh.at[pl.ds(ro, RSTG)])
    pltpu.sync_copy(n1_hbm.at[pl.ds(ro, RSTG)], s1_v.at[pl.ds(0, RSTG)])
    pltpu.sync_copy(s1_v.at[pl.ds(0, RSTG)], n1_sh.at[pl.ds(ro, RSTG)])

    def zero_body(i, _):
        sl = pl.ds(i * 16, 16)
        m0_v[sl] = fz
        m1_v[sl] = fz
        return 0
    lax.fori_loop(0, C // 16, zero_body, 0)

    @pl.when(cid == 0)
    def _():
        pltpu.sync_copy(s0_v.at[pl.ds(0, RSTG)], a0_sh.at[pl.ds(ro, RSTG)])
        pltpu.sync_copy(s1_v.at[pl.ds(0, RSTG)], a1_sh.at[pl.ds(ro, RSTG)])

    @pl.when(cid != 0)
    def _():
        pltpu.sync_copy(m0_v.at[pl.ds(0, RSTG)], a0_sh.at[pl.ds(ro, RSTG)])
        pltpu.sync_copy(m1_v.at[pl.ds(0, RSTG)], a1_sh.at[pl.ds(ro, RSTG)])

    pltpu.sync_copy(wb_hbm, wb_v)
    plsc.subcore_barrier()

    w00 = wb_v[pl.ds(0, 16)]
    w01 = wb_v[pl.ds(16, 16)]
    w10 = wb_v[pl.ds(32, 16)]
    w11 = wb_v[pl.ds(48, 16)]

    base = wid * EPW

    def chunk_body(k, _):
        off = base + k * C
        pltpu.sync_copy(src_hbm.at[pl.ds(off, C)], src_v)
        pltpu.sync_copy(tgt_hbm.at[pl.ds(off, C)], tgt_v)
        pltpu.sync_copy(wgt_hbm.at[pl.ds(off, C)], w_v)
        cp0 = pltpu.async_copy(n0_sh.at[src_v], s0_v, sem0)
        cp1 = pltpu.async_copy(n1_sh.at[src_v], s1_v, sem1)
        cp0.wait()
        cp1.wait()

        def vec_body(i, _):
            sl = pl.ds(i * 16, 16)
            s0 = s0_v[sl]
            s1 = s1_v[sl]
            w16 = w_v[sl]
            m0_v[sl] = w16 * (s0 * w00 + s1 * w10)
            m1_v[sl] = w16 * (s0 * w01 + s1 * w11)
            return 0
        lax.fori_loop(0, C // 16, vec_body, 0)

        pltpu.sync_copy(m0_v, a0_sh.at[tgt_v], add=True)
        pltpu.sync_copy(m1_v, a1_sh.at[tgt_v], add=True)
        return 0
    lax.fori_loop(0, NCHUNK, chunk_body, 0)

    plsc.subcore_barrier()

    oo = cid * N_PAD + ro
    pltpu.sync_copy(a0_sh.at[pl.ds(ro, RSTG)], s0_v.at[pl.ds(0, RSTG)])
    pltpu.sync_copy(s0_v.at[pl.ds(0, RSTG)], out0_hbm.at[pl.ds(oo, RSTG)])
    pltpu.sync_copy(a1_sh.at[pl.ds(ro, RSTG)], s1_v.at[pl.ds(0, RSTG)])
    pltpu.sync_copy(s1_v.at[pl.ds(0, RSTG)], out1_hbm.at[pl.ds(oo, RSTG)])


@functools.partial(
    pl.kernel,
    out_type=(
        jax.ShapeDtypeStruct((N_PAD,), jnp.float32),
        jax.ShapeDtypeStruct((N_PAD,), jnp.float32),
    ),
    mesh=_mesh,
    scratch_types=[
        pltpu.VMEM((NPW,), jnp.float32),
        pltpu.VMEM((NPW,), jnp.float32),
        pltpu.VMEM((NPW,), jnp.float32),
        pltpu.VMEM((NPW,), jnp.float32),
        pltpu.VMEM((NPW,), jnp.float32),
        pltpu.VMEM((NPW,), jnp.float32),
    ],
)
def _combine_kernel(p0_hbm, p1_hbm, out0_hbm, out1_hbm,
                    a0_v, a1_v, b0_v, b1_v, o0_v, o1_v):
    cid = lax.axis_index("c")
    sid = lax.axis_index("s")
    wid = cid * NS + sid
    off = wid * NPW
    pltpu.sync_copy(p0_hbm.at[pl.ds(off, NPW)], a0_v)
    pltpu.sync_copy(p1_hbm.at[pl.ds(off, NPW)], a1_v)
    pltpu.sync_copy(p0_hbm.at[pl.ds(N_PAD + off, NPW)], b0_v)
    pltpu.sync_copy(p1_hbm.at[pl.ds(N_PAD + off, NPW)], b1_v)

    def body(i, _):
        sl = pl.ds(i * 16, 16)
        o0_v[sl] = a0_v[sl] + b0_v[sl]
        o1_v[sl] = a1_v[sl] + b1_v[sl]
        return 0
    lax.fori_loop(0, NPW // 16, body, 0)
    pltpu.sync_copy(o0_v, out0_hbm.at[pl.ds(off, NPW)])
    pltpu.sync_copy(o1_v, out1_hbm.at[pl.ds(off, NPW)])


@jax.jit
def kernel(node_state, edge_weight, W, edge_source, edge_target):
    node_pad = jnp.zeros((N_PAD, 2), jnp.float32).at[:N_NODES].set(node_state)
    n0 = node_pad[:, 0]
    n1 = node_pad[:, 1]
    wf = W.reshape(-1)
    wb = jnp.concatenate([jnp.full((16,), wf[i], jnp.float32) for i in range(4)])
    wgt = edge_weight.reshape(-1)
    p0, p1 = _scatter_kernel(n0, n1, wb, edge_source, edge_target, wgt)
    o0, o1 = _combine_kernel(p0, p1)
    return jnp.stack((o0[:N_NODES], o1[:N_NODES]), axis=-1)


# speedup vs baseline: 156.5853x; 1.3233x over previous
import functools

import jax
import jax.numpy as jnp
from jax import lax
from jax.experimental import pallas as pl
from jax.experimental.pallas import tpu as pltpu
from jax.experimental.pallas import tpu_sc as plsc

N_NODES = 100000
N_PAD = 100352
N_EDGES = 6400000
NC, NS = 2, 16
NW = NC * NS
EPW = N_EDGES // NW
C = 2000
NCHUNK = EPW // C
RSTG = N_PAD // NS
HSTG = RSTG // 4
NPW = N_PAD // NW

_mesh = plsc.VectorSubcoreMesh(
    core_axis_name="c", subcore_axis_name="s", num_cores=NC, num_subcores=NS
)


@functools.partial(
    pl.kernel,
    out_type=(
        jax.ShapeDtypeStruct((NC * N_PAD,), jnp.float32),
        jax.ShapeDtypeStruct((NC * N_PAD,), jnp.float32),
    ),
    mesh=_mesh,
    scratch_types=[
        pltpu.VMEM_SHARED((N_PAD,), jnp.float32),
        pltpu.VMEM_SHARED((N_PAD,), jnp.float32),
        pltpu.VMEM_SHARED((N_PAD,), jnp.float32),
        pltpu.VMEM_SHARED((N_PAD,), jnp.float32),
        [pltpu.VMEM((C,), jnp.int32)] * 2,
        [pltpu.VMEM((C,), jnp.int32)] * 4,
        [pltpu.VMEM((C,), jnp.float32)] * 4,
        [pltpu.VMEM((C,), jnp.float32)] * 2,
        [pltpu.VMEM((C,), jnp.float32)] * 2,
        [pltpu.VMEM((C,), jnp.float32)] * 2,
        [pltpu.VMEM((C,), jnp.float32)] * 2,
        pltpu.VMEM((64,), jnp.float32),
        [pltpu.SemaphoreType.DMA] * 2,
        [pltpu.SemaphoreType.DMA] * 2,
        [pltpu.SemaphoreType.DMA] * 2,
    ],
)
def _scatter_kernel(n0_hbm, n1_hbm, wb_hbm, src_hbm, tgt_hbm, wgt_hbm,
                    out0_hbm, out1_hbm,
                    n0_sh, n1_sh, a0_sh, a1_sh,
                    src_v, tgt_v, w_v, s0_v, s1_v, m0_v, m1_v, wb_v,
                    sem_lin, sem_g, sem_s):
    cid = lax.axis_index("c")
    sid = lax.axis_index("s")
    wid = cid * NS + sid
    fz = jnp.zeros((16,), jnp.float32)

    ro = sid * RSTG
    for h in range(4):
        o = ro + h * HSTG
        pltpu.sync_copy(n0_hbm.at[pl.ds(o, HSTG)], s0_v[0].at[pl.ds(0, HSTG)])
        pltpu.sync_copy(s0_v[0].at[pl.ds(0, HSTG)], n0_sh.at[pl.ds(o, HSTG)])
        pltpu.sync_copy(n1_hbm.at[pl.ds(o, HSTG)], s1_v[0].at[pl.ds(0, HSTG)])
        pltpu.sync_copy(s1_v[0].at[pl.ds(0, HSTG)], n1_sh.at[pl.ds(o, HSTG)])

        @pl.when(cid == 0)
        def _():
            pltpu.sync_copy(s0_v[0].at[pl.ds(0, HSTG)], a0_sh.at[pl.ds(o, HSTG)])
            pltpu.sync_copy(s1_v[0].at[pl.ds(0, HSTG)], a1_sh.at[pl.ds(o, HSTG)])

    def zero_body(i, _):
        sl = pl.ds(i * 16, 16)
        m0_v[0][sl] = fz
        return 0
    lax.fori_loop(0, C // 16, zero_body, 0)

    @pl.when(cid != 0)
    def _():
        for h in range(4):
            o = ro + h * HSTG
            pltpu.sync_copy(m0_v[0].at[pl.ds(0, HSTG)], a0_sh.at[pl.ds(o, HSTG)])
            pltpu.sync_copy(m0_v[0].at[pl.ds(0, HSTG)], a1_sh.at[pl.ds(o, HSTG)])

    pltpu.sync_copy(wb_hbm, wb_v)
    plsc.subcore_barrier()

    w00 = wb_v[pl.ds(0, 16)]
    w01 = wb_v[pl.ds(16, 16)]
    w10 = wb_v[pl.ds(32, 16)]
    w11 = wb_v[pl.ds(48, 16)]

    base = wid * EPW

    def issue_lin(k, p, q):
        off = base + k * C
        pltpu.async_copy(src_hbm.at[pl.ds(off, C)], src_v[p], sem_lin[p])
        pltpu.async_copy(tgt_hbm.at[pl.ds(off, C)], tgt_v[q], sem_lin[p])
        pltpu.async_copy(wgt_hbm.at[pl.ds(off, C)], w_v[q], sem_lin[p])

    def wait_lin(p, q):
        pltpu.make_async_copy(src_hbm.at[pl.ds(0, C)], src_v[p], sem_lin[p]).wait()
        pltpu.make_async_copy(tgt_hbm.at[pl.ds(0, C)], tgt_v[q], sem_lin[p]).wait()
        pltpu.make_async_copy(wgt_hbm.at[pl.ds(0, C)], w_v[q], sem_lin[p]).wait()

    def issue_gather(p):
        pltpu.async_copy(n0_sh.at[src_v[p]], s0_v[p], sem_g[p])
        pltpu.async_copy(n1_sh.at[src_v[p]], s1_v[p], sem_g[p])

    def wait_gather(p):
        pltpu.make_async_copy(n0_sh.at[src_v[p]], s0_v[p], sem_g[p]).wait()
        pltpu.make_async_copy(n1_sh.at[src_v[p]], s1_v[p], sem_g[p]).wait()

    def issue_scatter(p, q):
        pltpu.async_copy(m0_v[p], a0_sh.at[tgt_v[q]], sem_s[p], add=True)
        pltpu.async_copy(m1_v[p], a1_sh.at[tgt_v[q]], sem_s[p], add=True)

    def wait_scatter(p, q):
        pltpu.make_async_copy(m0_v[p], a0_sh.at[tgt_v[q]], sem_s[p]).wait()
        pltpu.make_async_copy(m1_v[p], a1_sh.at[tgt_v[q]], sem_s[p]).wait()

    def compute(p, q):
        def vec_body(i, _):
            sl = pl.ds(i * 16, 16)
            s0 = s0_v[p][sl]
            s1 = s1_v[p][sl]
            w16 = w_v[q][sl]
            m0_v[p][sl] = w16 * (s0 * w00 + s1 * w10)
            m1_v[p][sl] = w16 * (s0 * w01 + s1 * w11)
            return 0
        lax.fori_loop(0, C // 16, vec_body, 0)

    issue_lin(0, 0, 0)
    issue_lin(1, 1, 1)
    wait_lin(0, 0)
    issue_gather(0)

    NSUP = NCHUNK // 4

    def super_body(j, _):
        for t in range(4):
            p, q = t % 2, t
            p1 = 1 - p
            q1, q2 = (t + 1) % 4, (t + 2) % 4
            wait_gather(p)

            if t < 3:
                wait_lin(p1, q1)
                issue_gather(p1)
            else:
                @pl.when(j < NSUP - 1)
                def _():
                    wait_lin(p1, q1)
                    issue_gather(p1)

            if t < 2:
                @pl.when(j >= 1)
                def _():
                    wait_scatter(p, q2)
            else:
                wait_scatter(p, q2)

            if t < 2:
                issue_lin(4 * j + t + 2, p, q2)
            else:
                @pl.when(j < NSUP - 1)
                def _():
                    issue_lin(4 * j + t + 2, p, q2)

            compute(p, q)
            issue_scatter(p, q)
        return 0
    lax.fori_loop(0, NSUP, super_body, 0)

    wait_scatter(0, 2)
    wait_scatter(1, 3)

    plsc.subcore_barrier()

    for h in range(4):
        o = ro + h * HSTG
        oo = cid * N_PAD + o
        pltpu.sync_copy(a0_sh.at[pl.ds(o, HSTG)], s0_v[0].at[pl.ds(0, HSTG)])
        pltpu.sync_copy(s0_v[0].at[pl.ds(0, HSTG)], out0_hbm.at[pl.ds(oo, HSTG)])
        pltpu.sync_copy(a1_sh.at[pl.ds(o, HSTG)], s1_v[0].at[pl.ds(0, HSTG)])
        pltpu.sync_copy(s1_v[0].at[pl.ds(0, HSTG)], out1_hbm.at[pl.ds(oo, HSTG)])


@functools.partial(
    pl.kernel,
    out_type=(
        jax.ShapeDtypeStruct((N_PAD,), jnp.float32),
        jax.ShapeDtypeStruct((N_PAD,), jnp.float32),
    ),
    mesh=_mesh,
    scratch_types=[
        pltpu.VMEM((NPW,), jnp.float32),
        pltpu.VMEM((NPW,), jnp.float32),
        pltpu.VMEM((NPW,), jnp.float32),
        pltpu.VMEM((NPW,), jnp.float32),
        pltpu.VMEM((NPW,), jnp.float32),
        pltpu.VMEM((NPW,), jnp.float32),
    ],
)
def _combine_kernel(p0_hbm, p1_hbm, out0_hbm, out1_hbm,
                    a0_v, a1_v, b0_v, b1_v, o0_v, o1_v):
    cid = lax.axis_index("c")
    sid = lax.axis_index("s")
    wid = cid * NS + sid
    off = wid * NPW
    pltpu.sync_copy(p0_hbm.at[pl.ds(off, NPW)], a0_v)
    pltpu.sync_copy(p1_hbm.at[pl.ds(off, NPW)], a1_v)
    pltpu.sync_copy(p0_hbm.at[pl.ds(N_PAD + off, NPW)], b0_v)
    pltpu.sync_copy(p1_hbm.at[pl.ds(N_PAD + off, NPW)], b1_v)

    def body(i, _):
        sl = pl.ds(i * 16, 16)
        o0_v[sl] = a0_v[sl] + b0_v[sl]
        o1_v[sl] = a1_v[sl] + b1_v[sl]
        return 0
    lax.fori_loop(0, NPW // 16, body, 0)
    pltpu.sync_copy(o0_v, out0_hbm.at[pl.ds(off, NPW)])
    pltpu.sync_copy(o1_v, out1_hbm.at[pl.ds(off, NPW)])


@jax.jit
def kernel(node_state, edge_weight, W, edge_source, edge_target):
    node_pad = jnp.zeros((N_PAD, 2), jnp.float32).at[:N_NODES].set(node_state)
    n0 = node_pad[:, 0]
    n1 = node_pad[:, 1]
    wf = W.reshape(-1)
    wb = jnp.concatenate([jnp.full((16,), wf[i], jnp.float32) for i in range(4)])
    wgt = edge_weight.reshape(-1)
    p0, p1 = _scatter_kernel(n0, n1, wb, edge_source, edge_target, wgt)
    o0, o1 = _combine_kernel(p0, p1)
    return jnp.stack((o0[:N_NODES], o1[:N_NODES]), axis=-1)
